# Initial kernel scaffold; baseline (speedup 1.0000x reference)
import jax
import jax.numpy as jnp
from jax.experimental import pallas as pl


def kernel(feature_map, rois):
    raise NotImplementedError("write your pallas kernel here")


# speedup vs baseline: 10.6234x; 10.6234x over previous
import functools

import jax
import jax.numpy as jnp
import numpy as np
from jax import lax
from jax.experimental import pallas as pl
from jax.experimental.pallas import tpu as pltpu
from jax.experimental.pallas import tpu_sc as plsc

OUT_H = 7
OUT_W = 7
SR = 2
S = OUT_H * SR
NP = 2 * S
NPIX = NP * NP
NPIX_PAD = 896
H = 224
W = 224
C = 192
CP = 256
LANES = 16
CB = C // LANES

NC = 2
NS = 16
NW = NC * NS
NROI = 1024
RPW = NROI // NW
CHUNK = 112
NCHUNK = NPIX // CHUNK
QPC = CHUNK // 8
WEXP = NPIX // 8 * 128

ROI_BLK = 64

_t = np.arange(WEXP)
_r = (_t // 128) * 8 + (_t % 128) // 16
_OHYW = (np.arange(NP)[:, None] == (_r // NP)[None, :]).astype(np.float32)
_OHXW = (np.arange(NP)[:, None] == (_r % NP)[None, :]).astype(np.float32)
_r2 = np.arange(NPIX_PAD)
_OHYI = (np.arange(32)[:, None] == (_r2 // NP)[None, :]).astype(np.float32)
_OHXI = (np.arange(NP)[:, None] == np.minimum(_r2 % NP, NP - 1)[None, :]
         ).astype(np.float32)


def _axis_pairs(t, lim):
    v = ((t >= -1.0) & (t <= lim)).astype(jnp.float32)
    t = jnp.maximum(t, 0.0)
    lim_i = int(lim)
    lo = jnp.minimum(jnp.floor(t).astype(jnp.int32), lim_i - 1)
    hi = jnp.minimum(lo + 1, lim_i - 1)
    t = jnp.where(lo >= lim_i - 1, lo.astype(jnp.float32), t)
    frac = t - lo.astype(jnp.float32)
    scale = 1.0 / S
    wlo = (1.0 - frac) * v * scale
    whi = frac * v * scale
    idx = jnp.concatenate([lo, hi], axis=1)
    wgt = jnp.concatenate([wlo, whi], axis=1)
    return idx, wgt


def _weights_body(boxes_ref, ohyw_ref, ohxw_ref, ohyi_ref, ohxi_ref,
                  wexp_ref, idx_ref):
    b = boxes_ref[...]
    x1 = b[:, 0]
    y1 = b[:, 1]
    x2 = b[:, 2]
    y2 = b[:, 3]
    roi_w = jnp.maximum(x2 - x1, 1.0)
    roi_h = jnp.maximum(y2 - y1, 1.0)
    bin_h = roi_h / OUT_H
    bin_w = roi_w / OUT_W
    g_i = lax.broadcasted_iota(jnp.int32, (ROI_BLK, S), 1)
    g = (g_i.astype(jnp.float32) + 0.5) / SR
    ys = y1[:, None] + bin_h[:, None] * g
    xs = x1[:, None] + bin_w[:, None] * g
    yi, wy = _axis_pairs(ys, float(H))
    xi, wx = _axis_pairs(xs, float(W))
    hp = lax.Precision.HIGHEST
    wexp_ref[...] = (
        jnp.dot(wy, ohyw_ref[...], precision=hp)
        * jnp.dot(wx, ohxw_ref[...], precision=hp))
    yif = jnp.concatenate(
        [yi.astype(jnp.float32), jnp.zeros((ROI_BLK, 4), jnp.float32)], axis=1)
    xif = xi.astype(jnp.float32)
    idxf = (jnp.dot(yif, ohyi_ref[...], precision=hp) * float(W)
            + jnp.dot(xif, ohxi_ref[...], precision=hp))
    idx_ref[...] = idxf.astype(jnp.int32)


_weights_tc = pl.pallas_call(
    _weights_body,
    grid=(NROI // ROI_BLK,),
    in_specs=[
        pl.BlockSpec((ROI_BLK, 4), lambda i: (i, 0)),
        pl.BlockSpec((NP, WEXP), lambda i: (0, 0)),
        pl.BlockSpec((NP, WEXP), lambda i: (0, 0)),
        pl.BlockSpec((32, NPIX_PAD), lambda i: (0, 0)),
        pl.BlockSpec((NP, NPIX_PAD), lambda i: (0, 0)),
    ],
    out_specs=[
        pl.BlockSpec((ROI_BLK, WEXP), lambda i: (i, 0)),
        pl.BlockSpec((ROI_BLK, NPIX_PAD), lambda i: (i, 0)),
    ],
    out_shape=[
        jax.ShapeDtypeStruct((NROI, WEXP), jnp.float32),
        jax.ShapeDtypeStruct((NROI, NPIX_PAD), jnp.int32),
    ],
)


def _sc_gather_body(table, idxs, ws, out, idx_v, w_v, rows_v, out_v,
                    sem_a, sem_b):
    wid = lax.axis_index("s") * NC + lax.axis_index("c")
    sems = (sem_a, sem_b)

    def roi_body(r, carry):
        roi = wid * RPW + r
        pltpu.sync_copy(idxs.at[roi], idx_v)
        pltpu.sync_copy(ws.at[roi], w_v)
        pending = pltpu.async_copy(
            table.at[idx_v.at[pl.ds(0, CHUNK)]], rows_v.at[0], sems[0])
        acc = tuple(jnp.zeros((LANES,), jnp.float32) for _ in range(CB))
        for c in range(NCHUNK):
            cur = c % 2
            pending.wait()
            if c + 1 < NCHUNK:
                pending = pltpu.async_copy(
                    table.at[idx_v.at[pl.ds((c + 1) * CHUNK, CHUNK)]],
                    rows_v.at[(c + 1) % 2], sems[(c + 1) % 2])

            def q_body(q, acc_t, cur=cur, c=c):
                for j in range(8):
                    sp = w_v[pl.ds((c * QPC + q) * 128 + j * 16, LANES)]
                    acc_t = tuple(
                        acc_t[i]
                        + sp * rows_v[cur, q * 8 + j, pl.ds(i * LANES, LANES)]
                        for i in range(CB))
                return acc_t

            acc = lax.fori_loop(0, QPC, q_body, acc)
        for i in range(CB):
            out_v[pl.ds(i * LANES, LANES)] = acc[i]
        pltpu.sync_copy(out_v, out.at[roi])
        return carry

    lax.fori_loop(0, RPW, roi_body, 0)


@functools.cache
def _make_sc_gather():
    return pl.kernel(
        _sc_gather_body,
        out_type=jax.ShapeDtypeStruct((NROI, CP), jnp.float32),
        mesh=plsc.VectorSubcoreMesh(
            core_axis_name="c", subcore_axis_name="s",
            num_cores=NC, num_subcores=NS),
        scratch_types=[
            pltpu.VMEM((NPIX_PAD,), jnp.int32),
            pltpu.VMEM((WEXP,), jnp.float32),
            pltpu.VMEM((2, CHUNK, CP), jnp.float32),
            pltpu.VMEM((CP,), jnp.float32),
            pltpu.SemaphoreType.DMA,
            pltpu.SemaphoreType.DMA,
        ],
    )


def kernel(feature_map, rois):
    fm = feature_map[0]
    table = jnp.pad(
        jnp.transpose(fm, (1, 2, 0)).reshape(H * W, C),
        ((0, 0), (0, CP - C)))
    gt = lax.stop_gradient(rois[:, 1])
    n = rois.shape[0]
    hf = float(feature_map.shape[2])
    wf = float(feature_map.shape[3])
    x1 = (rois[:, 2] - 0.5 * rois[:, 4]) * hf
    y1 = (rois[:, 3] - 0.5 * rois[:, 5]) * wf
    x2 = (rois[:, 2] + 0.5 * rois[:, 4]) * hf
    y2 = (rois[:, 3] + 0.5 * rois[:, 5]) * wf
    boxes = jnp.stack([x1, y1, x2, y2], axis=1)
    boxes = boxes.astype(jnp.float16).astype(jnp.float32)
    boxes = jnp.pad(boxes, ((0, NROI - n), (0, 0)))
    wexp, idx = _weights_tc(
        boxes, jnp.asarray(_OHYW), jnp.asarray(_OHXW),
        jnp.asarray(_OHYI), jnp.asarray(_OHXI))
    pooled = _make_sc_gather()(table, idx, wexp)
    return pooled[:n, :C], gt
